# baseline (speedup 1.0000x reference)
import functools

import jax
import jax.numpy as jnp
from jax import lax
from jax.experimental import pallas as pl
from jax.experimental.pallas import tpu as pltpu
from jax.experimental.pallas import tpu_sc as plsc

NC, NS, L = 2, 16, 16
NW = NC * NS
B = 16384
H = 64
W = 2 * H
BPW = B // NW
NCH = 4
CH = BPW // NCH
BLKS_PER_CH = CH // L

_MESH = plsc.VectorSubcoreMesh(core_axis_name="c", subcore_axis_name="s")


def _mf_body(users, items, user_emb, item_emb, user_bias, item_bias, out,
             idx_u, idx_v, pidx_u, pidx_v, rows_u, rows_v, bu, bv, out_v, sem):
    wid = lax.axis_index("s") * NC + lax.axis_index("c")

    pltpu.sync_copy(users.at[wid], idx_u)
    pltpu.sync_copy(items.at[wid], idx_v)

    for k in range(NCH):
        for m in range(CH // L):
            sl = pl.ds(m * L, L)
            pidx_u[k, sl] = lax.shift_right_logical(idx_u[k, sl], 1)
            pidx_v[k, sl] = lax.shift_right_logical(idx_v[k, sl], 1)

    bias_copies = []
    for k in range(NCH):
        bias_copies.append(pltpu.async_copy(user_bias.at[idx_u.at[k]], bu.at[k], sem))
        bias_copies.append(pltpu.async_copy(item_bias.at[idx_v.at[k]], bv.at[k], sem))
    for c in bias_copies:
        c.wait()

    for k in range(NCH):
        cu = pltpu.async_copy(user_emb.at[pidx_u.at[k]], rows_u, sem)
        cv = pltpu.async_copy(item_emb.at[pidx_v.at[k]], rows_v, sem)
        cu.wait()
        cv.wait()

        def blk(m, carry, k=k):
            iota = lax.iota(jnp.int32, L)
            one_hot = [(iota == i).astype(jnp.float32) for i in range(L)]
            rb = m * L
            acc = bu[k, pl.ds(rb, L)] + bv[k, pl.ds(rb, L)]
            offv_u = (idx_u[k, pl.ds(rb, L)] & 1) * H
            offv_v = (idx_v[k, pl.ds(rb, L)] & 1) * H
            for i in range(L):
                r = rb + i
                off_u = offv_u[i]
                off_v = offv_v[i]
                s = rows_u[r, pl.ds(off_u, L)] * rows_v[r, pl.ds(off_v, L)]
                for j in range(1, H // L):
                    s = s + (rows_u[r, pl.ds(off_u + j * L, L)]
                             * rows_v[r, pl.ds(off_v + j * L, L)])
                acc = acc + jnp.sum(s) * one_hot[i]
            out_v[pl.ds(k * CH + rb, L)] = acc
            return carry

        lax.fori_loop(0, BLKS_PER_CH, blk, 0)

    pltpu.sync_copy(out_v, out.at[wid])


_mf_sc = functools.partial(
    pl.kernel,
    out_type=jax.ShapeDtypeStruct((NW, BPW), jnp.float32),
    mesh=_MESH,
    compiler_params=pltpu.CompilerParams(needs_layout_passes=False),
    scratch_types=[
        pltpu.VMEM((NCH, CH), jnp.int32),
        pltpu.VMEM((NCH, CH), jnp.int32),
        pltpu.VMEM((NCH, CH), jnp.int32),
        pltpu.VMEM((NCH, CH), jnp.int32),
        pltpu.VMEM((CH, W), jnp.float32),
        pltpu.VMEM((CH, W), jnp.float32),
        pltpu.VMEM((NCH, CH), jnp.float32),
        pltpu.VMEM((NCH, CH), jnp.float32),
        pltpu.VMEM((BPW,), jnp.float32),
        pltpu.SemaphoreType.DMA,
    ],
)(_mf_body)


def kernel(users, items, user_emb, item_emb, user_bias, item_bias):
    users2 = users.reshape(NW, NCH, CH)
    items2 = items.reshape(NW, NCH, CH)
    u128 = user_emb.reshape(-1, W)
    i128 = item_emb.reshape(-1, W)
    out = _mf_sc(users2, items2, u128, i128, user_bias, item_bias)
    return out.reshape(B)


# speedup vs baseline: 1.7039x; 1.7039x over previous
import functools

import jax
import jax.numpy as jnp
from jax import lax
from jax.experimental import pallas as pl
from jax.experimental.pallas import tpu as pltpu
from jax.experimental.pallas import tpu_sc as plsc

NC, NS, L = 2, 16, 16
NW = NC * NS
B = 16384
H = 64
BPW = B // NW
NCH = 4
CH = BPW // NCH
NBLK = BPW // L

_MESH = plsc.VectorSubcoreMesh(core_axis_name="c", subcore_axis_name="s")


def _mf_body(users, items, user_emb, item_emb, user_bias, item_bias, out,
             idx_u, idx_v, rows_u, rows_v, bu, bv, out_v, sem, sem_u, sem_v):
    wid = lax.axis_index("s") * NC + lax.axis_index("c")

    pltpu.sync_copy(users.at[wid], idx_u)
    pltpu.sync_copy(items.at[wid], idx_v)

    bias_copies = []
    for k in range(NCH):
        bias_copies.append(pltpu.async_copy(user_bias.at[idx_u.at[k]], bu.at[k], sem))
        bias_copies.append(pltpu.async_copy(item_bias.at[idx_v.at[k]], bv.at[k], sem))

    def fire(b, carry):
        k = b // (CH // L)
        rb = (b % (CH // L)) * L
        iu = idx_u[k, pl.ds(rb, L)]
        iv = idx_v[k, pl.ds(rb, L)]
        for i in range(L):
            slot = b * (L // 2) + i // 2
            half = pl.ds((i % 2) * H, H)
            pltpu.async_copy(user_emb.at[iu[i]], rows_u.at[slot, half], sem_u)
            pltpu.async_copy(item_emb.at[iv[i]], rows_v.at[slot, half], sem_v)
        return carry

    lax.fori_loop(0, NBLK, fire, 0)

    for t in range(BPW // 2 // 4):
        pltpu.make_async_copy(out.at[wid], rows_u.at[pl.ds(t * 4, 4)], sem_u).wait()
        pltpu.make_async_copy(out.at[wid], rows_v.at[pl.ds(t * 4, 4)], sem_v).wait()
    for c in bias_copies:
        c.wait()

    def blk(m, carry):
        iota = lax.iota(jnp.int32, L)
        one_hot = [(iota == i).astype(jnp.float32) for i in range(L)]
        k = m // (CH // L)
        rb = (m % (CH // L)) * L
        acc = bu[k, pl.ds(rb, L)] + bv[k, pl.ds(rb, L)]
        for i in range(L):
            slot = m * (L // 2) + i // 2
            half = (i % 2) * H
            s = (rows_u[slot, pl.ds(half, L)] * rows_v[slot, pl.ds(half, L)])
            for j in range(1, H // L):
                s = s + (rows_u[slot, pl.ds(half + j * L, L)]
                         * rows_v[slot, pl.ds(half + j * L, L)])
            acc = acc + jnp.sum(s) * one_hot[i]
        out_v[m // 8, pl.ds((m % 8) * L, L)] = acc
        return carry

    lax.fori_loop(0, NBLK, blk, 0)

    pltpu.sync_copy(out_v, out.at[wid])


_mf_sc = functools.partial(
    pl.kernel,
    out_type=jax.ShapeDtypeStruct((NW, BPW // 128, 128), jnp.float32),
    mesh=_MESH,
    compiler_params=pltpu.CompilerParams(needs_layout_passes=False),
    scratch_types=[
        pltpu.VMEM((NCH, CH), jnp.int32),
        pltpu.VMEM((NCH, CH), jnp.int32),
        pltpu.VMEM((BPW // 2, 2 * H), jnp.float32),
        pltpu.VMEM((BPW // 2, 2 * H), jnp.float32),
        pltpu.VMEM((NCH, CH), jnp.float32),
        pltpu.VMEM((NCH, CH), jnp.float32),
        pltpu.VMEM((BPW // 128, 128), jnp.float32),
        pltpu.SemaphoreType.DMA,
        pltpu.SemaphoreType.DMA,
        pltpu.SemaphoreType.DMA,
    ],
)(_mf_body)


def kernel(users, items, user_emb, item_emb, user_bias, item_bias):
    users2 = users.reshape(NW, NCH, CH)
    items2 = items.reshape(NW, NCH, CH)
    out = _mf_sc(users2, items2, user_emb, item_emb, user_bias, item_bias)
    return out.reshape(B)
